# baseline (speedup 1.0000x reference)
import functools

import jax
import jax.numpy as jnp
from jax import lax
from jax.experimental import pallas as pl
from jax.experimental.pallas import tpu as pltpu
from jax.experimental.pallas import tpu_sc as plsc

NC = 2
NS = 16
NW = NC * NS

B = 16384
C = 26
D = 32
L = 16
BPW = B // NW
GRP = 128
GPS = BPW // GRP
STEPS = C * GPS


def _sc_gather(x_t, table):
    mesh = plsc.VectorSubcoreMesh(core_axis_name="c", subcore_axis_name="s")

    @functools.partial(
        pl.kernel,
        mesh=mesh,
        out_type=jax.ShapeDtypeStruct((C, D // 8, B // GRP, 8, GRP), jnp.float32),
        compiler_params=pltpu.CompilerParams(
            use_tc_tiling_on_sc=False, needs_layout_passes=False
        ),
        scratch_types=[
            pltpu.VMEM((C, BPW), jnp.int32),
            pltpu.VMEM((GRP, D), jnp.float32),
            pltpu.VMEM((GRP, D), jnp.float32),
            pltpu.VMEM((D // 8, 8, GRP), jnp.float32),
            pltpu.VMEM((D // 8, 8, GRP), jnp.float32),
            pltpu.SemaphoreType.DMA,
            pltpu.SemaphoreType.DMA,
            pltpu.SemaphoreType.DMA,
            pltpu.SemaphoreType.DMA,
        ],
    )
    def k(xt_hbm, table_hbm, out_hbm, idx_v, gb0, gb1, tb0, tb1, g0, g1, o0, o1):
        wid = lax.axis_index("s") * NC + lax.axis_index("c")
        base = wid * BPW
        pltpu.sync_copy(xt_hbm.at[:, pl.ds(base, BPW)], idx_v)

        gbufs = (gb0, gb1)
        tbufs = (tb0, tb1)
        gsem = (g0, g1)
        osem = (o0, o1)
        lane = lax.broadcasted_iota(jnp.int32, (L,), 0)

        def islice(s):
            c = s // GPS
            blk = s % GPS
            return idx_v.at[c, pl.ds(blk * GRP, GRP)]

        def fire(s, p):
            pltpu.async_copy(table_hbm.at[islice(s)], gbufs[p], gsem[p])

        def drain(s, p):
            pltpu.make_async_copy(table_hbm.at[islice(s)], gbufs[p], gsem[p]).wait()

        def transpose(p):
            gb, tb = gbufs[p], tbufs[p]
            for d in range(D):
                for l0 in range(0, GRP, L):
                    v = plsc.load_gather(
                        gb, [lane + l0, jnp.full((L,), d, jnp.int32)]
                    )
                    tb[d // 8, d % 8, pl.ds(l0, L)] = v

        def out_dst(s):
            c = s // GPS
            tc = wid * GPS + s % GPS
            return out_hbm.at[c, :, tc]

        def out_fire(s, p):
            pltpu.async_copy(tbufs[p], out_dst(s), osem[p])

        def out_wait(s, p):
            pltpu.make_async_copy(tbufs[p], out_dst(s), osem[p]).wait()

        fire(0, 0)

        def body(i, carry):
            s = i * 2
            fire(s + 1, 1)
            drain(s, 0)
            pl.when(s >= 2)(lambda: out_wait(s - 2, 0))
            transpose(0)
            out_fire(s, 0)
            pl.when(s + 2 < STEPS)(lambda: fire(s + 2, 0))
            drain(s + 1, 1)
            pl.when(s >= 1)(lambda: out_wait(s - 1, 1))
            transpose(1)
            out_fire(s + 1, 1)
            return carry

        lax.fori_loop(0, STEPS // 2, body, 0)
        out_wait(STEPS - 2, 0)
        out_wait(STEPS - 1, 1)

    return k(x_t, table)


def kernel(x, table):
    out5 = _sc_gather(x.T, table)
    return out5.transpose(2, 4, 0, 1, 3).reshape(B, C, D)


# speedup vs baseline: 1.4325x; 1.4325x over previous
import functools

import jax
import jax.numpy as jnp
from jax import lax
from jax.experimental import pallas as pl
from jax.experimental.pallas import tpu as pltpu
from jax.experimental.pallas import tpu_sc as plsc

NC = 2
NS = 16
NW = NC * NS

B = 16384
C = 26
D = 32
L = 16
BPW = B // NW
GRP = 128
GPS = BPW // GRP
STEPS = C * GPS


def _sc_gather(x_t, table):
    mesh = plsc.VectorSubcoreMesh(core_axis_name="c", subcore_axis_name="s")

    @functools.partial(
        pl.kernel,
        mesh=mesh,
        out_type=jax.ShapeDtypeStruct((C, D // 8, B // GRP, 8, GRP), jnp.float32),
        compiler_params=pltpu.CompilerParams(
            use_tc_tiling_on_sc=False, needs_layout_passes=False
        ),
        scratch_types=[
            pltpu.VMEM((C, BPW), jnp.int32),
            pltpu.VMEM((GRP, D), jnp.float32),
            pltpu.VMEM((GRP, D), jnp.float32),
            pltpu.VMEM((D // 8, 8, GRP + 1), jnp.float32),
            pltpu.VMEM((D // 8, 8, GRP + 1), jnp.float32),
            pltpu.SemaphoreType.DMA,
            pltpu.SemaphoreType.DMA,
            pltpu.SemaphoreType.DMA,
            pltpu.SemaphoreType.DMA,
        ],
    )
    def k(xt_hbm, table_hbm, out_hbm, idx_v, gb0, gb1, tb0, tb1, g0, g1, o0, o1):
        wid = lax.axis_index("s") * NC + lax.axis_index("c")
        base = wid * BPW
        pltpu.sync_copy(xt_hbm.at[:, pl.ds(base, BPW)], idx_v)

        gbufs = (gb0, gb1)
        tbufs = (tb0, tb1)
        gsem = (g0, g1)
        osem = (o0, o1)
        d_lo = lax.broadcasted_iota(jnp.int32, (L,), 0)
        tr_lo, r_lo = d_lo // 8, d_lo % 8
        d_hi = d_lo + L
        tr_hi, r_hi = d_hi // 8, d_hi % 8

        def islice(s):
            c = s // GPS
            blk = s % GPS
            return idx_v.at[c, pl.ds(blk * GRP, GRP)]

        def fire(s, p):
            pltpu.async_copy(table_hbm.at[islice(s)], gbufs[p], gsem[p])

        def drain(s, p):
            pltpu.make_async_copy(table_hbm.at[islice(s)], gbufs[p], gsem[p]).wait()

        def transpose(p):
            gb, tb = gbufs[p], tbufs[p]
            for l in range(GRP):
                lv = jnp.full((L,), l, jnp.int32)
                v0 = gb[l, pl.ds(0, L)]
                v1 = gb[l, pl.ds(L, L)]
                plsc.store_scatter(tb, [tr_lo, r_lo, lv], v0)
                plsc.store_scatter(tb, [tr_hi, r_hi, lv], v1)

        def out_dst(s):
            c = s // GPS
            tc = wid * GPS + s % GPS
            return out_hbm.at[c, :, tc]

        def out_fire(s, p):
            pltpu.async_copy(
                tbufs[p].at[:, :, pl.ds(0, GRP)], out_dst(s), osem[p]
            )

        def out_wait(s, p):
            pltpu.make_async_copy(
                tbufs[p].at[:, :, pl.ds(0, GRP)], out_dst(s), osem[p]
            ).wait()

        fire(0, 0)

        def body(i, carry):
            s = i * 2
            fire(s + 1, 1)
            drain(s, 0)
            pl.when(s >= 2)(lambda: out_wait(s - 2, 0))
            transpose(0)
            out_fire(s, 0)
            pl.when(s + 2 < STEPS)(lambda: fire(s + 2, 0))
            drain(s + 1, 1)
            pl.when(s >= 1)(lambda: out_wait(s - 1, 1))
            transpose(1)
            out_fire(s + 1, 1)
            return carry

        lax.fori_loop(0, STEPS // 2, body, 0)
        out_wait(STEPS - 2, 0)
        out_wait(STEPS - 1, 1)

    return k(x_t, table)


def kernel(x, table):
    out5 = _sc_gather(x.T, table)
    return out5.transpose(2, 4, 0, 1, 3).reshape(B, C, D)
